# baseline (speedup 1.0000x reference)
import functools

import jax
import jax.numpy as jnp
from jax import lax
from jax.experimental import pallas as pl
from jax.experimental.pallas import tpu as pltpu
from jax.experimental.pallas import tpu_sc as plsc

NUM_CORES = 2
NUM_SUBCORES = 16
NUM_WORKERS = NUM_CORES * NUM_SUBCORES
LANES = 16
CHUNK = 64


@functools.lru_cache(maxsize=None)
def _build(n, b, d):
    assert b % (8 * NUM_WORKERS) == 0
    b_per_w = b // NUM_WORKERS
    assert b_per_w % CHUNK == 0 and b_per_w % (LANES * LANES) == 0
    n_chunks = b_per_w // CHUNK
    n_groups = b_per_w // LANES
    n_halves = n_groups // LANES
    mesh = plsc.VectorSubcoreMesh(
        core_axis_name="c", subcore_axis_name="s",
        num_cores=NUM_CORES, num_subcores=NUM_SUBCORES)

    @functools.partial(
        pl.kernel,
        out_type=jax.ShapeDtypeStruct((b, d), jnp.float32),
        mesh=mesh,
        scratch_types=[
            pltpu.VMEM((b,), jnp.int32),
            pltpu.VMEM((b,), jnp.int32),
            pltpu.VMEM((b_per_w,), jnp.int32),
            pltpu.VMEM((b_per_w,), jnp.int32),
            pltpu.VMEM((b_per_w,), jnp.int32),
            pltpu.VMEM((CHUNK, d), jnp.float32),
            pltpu.VMEM((CHUNK, d), jnp.float32),
            pltpu.VMEM((LANES, d), jnp.float32),
            pltpu.VMEM_SHARED((n,), jnp.int32),
            pltpu.SemaphoreType.DMA,
            pltpu.SemaphoreType.DMA,
            pltpu.SemaphoreType.DMA,
        ],
    )
    def bank(ids_hbm, iota_hbm, upd_hbm, out_hbm,
             ids_all_v, iota_v, ids_v, w_v, diff_v, rows_a, rows_b, fix_v,
             table_sh, sem_a, sem_b, sem_c):
        c = lax.axis_index("c")
        s = lax.axis_index("s")
        wid = c * NUM_SUBCORES + s
        base = wid * b_per_w

        own = pltpu.async_copy(
            ids_hbm.at[pl.ds(base, b_per_w)], ids_v, sem_c)

        @pl.when(s == 0)
        def _table_build():
            pltpu.sync_copy(ids_hbm, ids_all_v)
            pltpu.sync_copy(iota_hbm, iota_v)
            pltpu.sync_copy(iota_v, table_sh.at[ids_all_v])

        bufs = (rows_a, rows_b)
        sems = (sem_a, sem_b)
        handles = [None] * n_chunks
        handles[0] = pltpu.async_copy(
            upd_hbm.at[pl.ds(base, CHUNK)], bufs[0], sems[0])
        for k in range(n_chunks):
            if k + 1 < n_chunks:
                handles[k + 1] = pltpu.async_copy(
                    upd_hbm.at[pl.ds(base + (k + 1) * CHUNK, CHUNK)],
                    bufs[(k + 1) % 2], sems[(k + 1) % 2])
            handles[k].wait()
            pltpu.sync_copy(
                bufs[k % 2], out_hbm.at[pl.ds(base + k * CHUNK, CHUNK)])

        own.wait()
        plsc.subcore_barrier()

        pltpu.sync_copy(table_sh.at[ids_v], w_v)

        lane = lax.iota(jnp.int32, LANES)
        perms = [lane ^ (1 << k) for k in range(4)]
        for g in range(n_groups):
            pos = lane + (base + g * LANES)
            v = w_v[pl.ds(g * LANES, LANES)] ^ pos
            for perm in perms:
                v = v | v.at[perm].get(mode="promise_in_bounds")

            @pl.when(v[0] != 0)
            def _fixup(g=g):
                pltpu.async_copy(
                    upd_hbm.at[w_v.at[pl.ds(g * LANES, LANES)]],
                    fix_v, sem_c).wait()
                pltpu.sync_copy(
                    fix_v, out_hbm.at[pl.ds(base + g * LANES, LANES)])

    return bank


def kernel(node_memories, node_ids, updated_node_memories):
    n = node_memories.shape[0]
    b, d = updated_node_memories.shape
    ids = node_ids.astype(jnp.int32)
    iota = jnp.arange(b, dtype=jnp.int32)
    return _build(n, b, d)(ids, iota, updated_node_memories)


# speedup vs baseline: 1.3555x; 1.3555x over previous
import functools

import jax
import jax.numpy as jnp
from jax import lax
from jax.experimental import pallas as pl
from jax.experimental.pallas import tpu as pltpu
from jax.experimental.pallas import tpu_sc as plsc

NUM_CORES = 2
NUM_SUBCORES = 16
NUM_WORKERS = NUM_CORES * NUM_SUBCORES
CHUNK = 128


@functools.lru_cache(maxsize=None)
def _build(n, b, d):
    assert b % (8 * NUM_WORKERS) == 0
    b_per_w = b // NUM_WORKERS
    assert b_per_w % CHUNK == 0
    n_chunks = b_per_w // CHUNK
    mesh = plsc.VectorSubcoreMesh(
        core_axis_name="c", subcore_axis_name="s",
        num_cores=NUM_CORES, num_subcores=NUM_SUBCORES)

    @functools.partial(
        pl.kernel,
        out_type=jax.ShapeDtypeStruct((b, d), jnp.float32),
        mesh=mesh,
        scratch_types=[
            pltpu.VMEM((b,), jnp.int32),
            pltpu.VMEM((b,), jnp.int32),
            pltpu.VMEM((b_per_w,), jnp.int32),
            pltpu.VMEM((b_per_w,), jnp.int32),
            pltpu.VMEM((CHUNK, d), jnp.float32),
            pltpu.VMEM((CHUNK, d), jnp.float32),
            pltpu.VMEM_SHARED((n,), jnp.int32),
            pltpu.SemaphoreType.DMA,
            pltpu.SemaphoreType.DMA,
            pltpu.SemaphoreType.DMA,
        ],
    )
    def bank(ids_hbm, iota_hbm, upd_hbm, out_hbm,
             ids_all_v, iota_v, ids_v, w_v, rows_a, rows_b, table_sh,
             sem_a, sem_b, sem_c):
        c = lax.axis_index("c")
        s = lax.axis_index("s")
        wid = c * NUM_SUBCORES + s
        base = wid * b_per_w

        own = pltpu.async_copy(
            ids_hbm.at[pl.ds(base, b_per_w)], ids_v, sem_c)

        @pl.when(s == 0)
        def _table_build():
            st1 = pltpu.async_copy(ids_hbm, ids_all_v, sem_a)
            st2 = pltpu.async_copy(iota_hbm, iota_v, sem_b)
            st1.wait()
            st2.wait()
            pltpu.sync_copy(iota_v, table_sh.at[ids_all_v])

        own.wait()
        plsc.subcore_barrier()

        pltpu.sync_copy(table_sh.at[ids_v], w_v)

        bufs = (rows_a, rows_b)
        sems = (sem_a, sem_b)
        handles = [None] * n_chunks
        handles[0] = pltpu.async_copy(
            upd_hbm.at[w_v.at[pl.ds(0, CHUNK)]], bufs[0], sems[0])
        for k in range(n_chunks):
            if k + 1 < n_chunks:
                handles[k + 1] = pltpu.async_copy(
                    upd_hbm.at[w_v.at[pl.ds((k + 1) * CHUNK, CHUNK)]],
                    bufs[(k + 1) % 2], sems[(k + 1) % 2])
            handles[k].wait()
            pltpu.sync_copy(
                bufs[k % 2], out_hbm.at[pl.ds(base + k * CHUNK, CHUNK)])

    return bank


def kernel(node_memories, node_ids, updated_node_memories):
    n = node_memories.shape[0]
    b, d = updated_node_memories.shape
    ids = node_ids.astype(jnp.int32)
    iota = jnp.arange(b, dtype=jnp.int32)
    return _build(n, b, d)(ids, iota, updated_node_memories)
